# Initial kernel scaffold; baseline (speedup 1.0000x reference)
import jax
import jax.numpy as jnp
from jax.experimental import pallas as pl


def kernel(x, emb1, emb2):
    raise NotImplementedError("write your pallas kernel here")


# speedup vs baseline: 6.5691x; 6.5691x over previous
import functools

import jax
import jax.numpy as jnp
import numpy as np
from jax import lax
from jax.experimental import pallas as pl
from jax.experimental.pallas import tpu as pltpu
from jax.experimental.pallas import tpu_sc as plsc

_L = 16


def _build_sc_kernel(n_idx, v_rows, d_cols, n_workers, chunk, stream_w):
    per_worker = n_idx // n_workers
    n_chunks = per_worker // chunk
    n_streams = chunk // stream_w
    mesh = plsc.VectorSubcoreMesh(core_axis_name="c", subcore_axis_name="s")

    @functools.partial(
        pl.kernel,
        mesh=mesh,
        compiler_params=pltpu.CompilerParams(use_tc_tiling_on_sc=False),
        out_type=jax.ShapeDtypeStruct((n_idx, d_cols), jnp.float32),
        scratch_types=[
            pltpu.VMEM((v_rows, _L), jnp.float32),
            pltpu.VMEM((v_rows, _L), jnp.float32),
            pltpu.VMEM((_L,), jnp.float32),
            pltpu.VMEM_SHARED((v_rows, d_cols), jnp.float32),
            pltpu.VMEM((chunk,), jnp.int32),
            pltpu.VMEM((chunk, d_cols), jnp.float32),
            pltpu.SemaphoreType.DMA,
        ],
    )
    def k(idx_hbm, t1_hbm, t2_hbm, out_hbm, t1_v, t2_v, crow_v, comb_sh,
          idx_v, rows_v, sem):
        cid = lax.axis_index("c")
        sid = lax.axis_index("s")
        wid = sid * 2 + cid

        @pl.when(sid == 0)
        def _build():
            pltpu.sync_copy(t1_hbm, t1_v)
            pltpu.sync_copy(t2_hbm, t2_v)
            for r in range(v_rows):
                crow_v[...] = t1_v[r, :] + t2_v[r, :]
                pltpu.sync_copy(crow_v.at[pl.ds(0, d_cols)], comb_sh.at[r])

        plsc.subcore_barrier()

        base = wid * per_worker

        def chunk_body(c, carry):
            row0 = base + c * chunk
            pltpu.sync_copy(idx_hbm.at[pl.ds(row0, chunk)], idx_v)

            def gather_body(j, carry2):
                src = comb_sh.at[idx_v.at[pl.ds(j * stream_w, stream_w)]]
                dst = rows_v.at[pl.ds(j * stream_w, stream_w), :]
                pltpu.async_copy(src, dst, sem).wait()
                return carry2

            lax.fori_loop(0, n_streams, gather_body, 0, unroll=False)
            pltpu.sync_copy(rows_v, out_hbm.at[pl.ds(row0, chunk)])
            return carry

        lax.fori_loop(0, n_chunks, chunk_body, 0, unroll=False)

    return k


@jax.jit
def kernel(x, emb1, emb2):
    b0, b1 = x.shape
    v_rows, d_cols = emb1.shape
    n_idx = b0 * b1
    xf = x.reshape(n_idx).astype(jnp.int32)
    e1p = jnp.pad(emb1, ((0, 0), (0, _L - d_cols)))
    e2p = jnp.pad(emb2, ((0, 0), (0, _L - d_cols)))
    k = _build_sc_kernel(n_idx, v_rows, d_cols,
                         n_workers=32, chunk=6400, stream_w=128)
    out = k(xf, e1p, e2p)
    return out.reshape(b0, b1, d_cols)


# speedup vs baseline: 6.5982x; 1.0044x over previous
import functools

import jax
import jax.numpy as jnp
import numpy as np
from jax import lax
from jax.experimental import pallas as pl
from jax.experimental.pallas import tpu as pltpu
from jax.experimental.pallas import tpu_sc as plsc

_L = 16


def _build_sc_kernel(n_idx, v_rows, d_cols, n_workers, chunk, stream_w):
    per_worker = n_idx // n_workers
    n_chunks = per_worker // chunk
    n_streams = chunk // stream_w
    mesh = plsc.VectorSubcoreMesh(core_axis_name="c", subcore_axis_name="s")

    @functools.partial(
        pl.kernel,
        mesh=mesh,
        compiler_params=pltpu.CompilerParams(use_tc_tiling_on_sc=False),
        out_type=jax.ShapeDtypeStruct((n_idx, d_cols), jnp.float32),
        scratch_types=[
            pltpu.VMEM((v_rows, _L), jnp.float32),
            pltpu.VMEM((v_rows, _L), jnp.float32),
            pltpu.VMEM((_L,), jnp.float32),
            pltpu.VMEM_SHARED((v_rows, d_cols), jnp.float32),
            pltpu.VMEM((chunk,), jnp.int32),
            pltpu.VMEM((chunk, d_cols), jnp.float32),
            pltpu.SemaphoreType.DMA,
        ],
    )
    def k(idx_hbm, t1_hbm, t2_hbm, out_hbm, t1_v, t2_v, crow_v, comb_sh,
          idx_v, rows_v, sem):
        cid = lax.axis_index("c")
        sid = lax.axis_index("s")
        wid = sid * 2 + cid

        @pl.when(sid == 0)
        def _build():
            pltpu.sync_copy(t1_hbm, t1_v)
            pltpu.sync_copy(t2_hbm, t2_v)
            for r in range(v_rows):
                crow_v[...] = t1_v[r, :] + t2_v[r, :]
                pltpu.sync_copy(crow_v.at[pl.ds(0, d_cols)], comb_sh.at[r])

        plsc.subcore_barrier()

        base = wid * per_worker

        def chunk_body(c, carry):
            row0 = base + c * chunk
            pltpu.sync_copy(idx_hbm.at[pl.ds(row0, chunk)], idx_v)
            group = 10
            for g in range(0, n_streams, group):
                descs = []
                for j in range(g, min(g + group, n_streams)):
                    src = comb_sh.at[idx_v.at[pl.ds(j * stream_w, stream_w)]]
                    dst = rows_v.at[pl.ds(j * stream_w, stream_w), :]
                    descs.append(pltpu.async_copy(src, dst, sem))
                for d in descs:
                    d.wait()
            pltpu.sync_copy(rows_v, out_hbm.at[pl.ds(row0, chunk)])
            return carry

        lax.fori_loop(0, n_chunks, chunk_body, 0, unroll=False)

    return k


@jax.jit
def kernel(x, emb1, emb2):
    b0, b1 = x.shape
    v_rows, d_cols = emb1.shape
    n_idx = b0 * b1
    xf = x.reshape(n_idx).astype(jnp.int32)
    e1p = jnp.pad(emb1, ((0, 0), (0, _L - d_cols)))
    e2p = jnp.pad(emb2, ((0, 0), (0, _L - d_cols)))
    k = _build_sc_kernel(n_idx, v_rows, d_cols,
                         n_workers=32, chunk=6400, stream_w=128)
    out = k(xf, e1p, e2p)
    return out.reshape(b0, b1, d_cols)


# speedup vs baseline: 44.6952x; 6.7739x over previous
import functools

import jax
import jax.numpy as jnp
from jax import lax
from jax.experimental import pallas as pl
from jax.experimental.pallas import tpu as pltpu
from jax.experimental.pallas import tpu_sc as plsc

_L = 16


def _build_sc_kernel(n_i, n_j, v_rows, d_cols):
    n_it = n_i // 128
    n_jt = (n_j + 7) // 8
    n_workers = 32
    it_per_w = n_it // n_workers
    grp = 4
    mesh = plsc.VectorSubcoreMesh(core_axis_name="c", subcore_axis_name="s")

    @functools.partial(
        pl.kernel,
        mesh=mesh,
        compiler_params=pltpu.CompilerParams(use_tc_tiling_on_sc=False,
                                             needs_layout_passes=False),
        out_type=jax.ShapeDtypeStruct((d_cols, n_jt, n_it, 8, 128),
                                      jnp.float32),
        scratch_types=[
            pltpu.VMEM((v_rows, _L), jnp.float32),
            pltpu.VMEM((v_rows, _L), jnp.float32),
            pltpu.VMEM((v_rows * _L,), jnp.float32),
            pltpu.VMEM((128 * n_j,), jnp.int32),
            pltpu.VMEM((d_cols, grp, 1, 8, 128), jnp.float32),
            pltpu.VMEM((d_cols, grp, 1, 8, 128), jnp.float32),
            pltpu.SemaphoreType.DMA,
            pltpu.SemaphoreType.DMA,
        ],
    )
    def k(idx_hbm, t1_hbm, t2_hbm, out_hbm, t1_v, t2_v, comb_v, idx_v,
          buf_a, buf_b, sem_a, sem_b):
        cid = lax.axis_index("c")
        sid = lax.axis_index("s")
        wid = sid * 2 + cid

        pltpu.sync_copy(t1_hbm, t1_v)
        pltpu.sync_copy(t2_hbm, t2_v)
        for r in range(v_rows):
            comb_v[pl.ds(r * _L, _L)] = t1_v[r, :] + t2_v[r, :]

        lanes100 = lax.iota(jnp.int32, _L) * n_j

        def do_group(it, jt0, njt, nj, buf):
            def j_body(jl, carry):
                jglob = jt0 * 8 + jl
                jtl = jl // 8 if njt > 1 else 0
                for r in range(8):
                    av = lanes100 + (r * _L * n_j + jglob)
                    xi = plsc.load_gather(idx_v, [av])
                    xi16 = xi * _L
                    for kk in range(d_cols):
                        val = plsc.load_gather(comb_v, [xi16 + kk])
                        buf[kk, jtl, 0, jl - jtl * 8, pl.ds(r * _L, _L)] = val
                return carry

            if njt > 1:
                for jtl in range(njt):
                    def jt_body(jin, carry, jtl=jtl):
                        jglob = (jt0 + jtl) * 8 + jin
                        for r in range(8):
                            av = lanes100 + (r * _L * n_j + jglob)
                            xi = plsc.load_gather(idx_v, [av])
                            xi16 = xi * _L
                            for kk in range(d_cols):
                                val = plsc.load_gather(comb_v, [xi16 + kk])
                                buf[kk, jtl, 0, jin, pl.ds(r * _L, _L)] = val
                        return carry
                    lax.fori_loop(0, 8, jt_body, 0, unroll=False)
            else:
                lax.fori_loop(0, nj, j_body, 0, unroll=False)
            descs = []
            for kk in range(d_cols):
                src = buf.at[kk, pl.ds(0, njt)]
                dst = out_hbm.at[kk, pl.ds(jt0, njt), pl.ds(it, 1)]
                descs.append(pltpu.async_copy(
                    src, dst, sem_a if buf is buf_a else sem_b))
            return descs

        def tile_body(t, carry):
            it = wid * it_per_w + t
            pltpu.sync_copy(idx_hbm.at[pl.ds(it * 128 * n_j, 128 * n_j)],
                            idx_v)
            d0 = do_group(it, 0, grp, 8 * grp, buf_a)
            d1 = do_group(it, grp, grp, 8 * grp, buf_b)
            for d in d0:
                d.wait()
            d2 = do_group(it, 2 * grp, grp, 8 * grp, buf_a)
            for d in d1:
                d.wait()
            d3 = do_group(it, 3 * grp, 1, n_j - 24 * grp, buf_b)
            for d in d2 + d3:
                d.wait()
            return carry

        lax.fori_loop(0, it_per_w, tile_body, 0, unroll=False)

    return k


@jax.jit
def kernel(x, emb1, emb2):
    n_i, n_j = x.shape
    v_rows, d_cols = emb1.shape
    xf = x.reshape(n_i * n_j).astype(jnp.int32)
    e1p = jnp.pad(emb1, ((0, 0), (0, _L - d_cols)))
    e2p = jnp.pad(emb2, ((0, 0), (0, _L - d_cols)))
    k = _build_sc_kernel(n_i, n_j, v_rows, d_cols)
    out5 = k(xf, e1p, e2p)
    n_jt = (n_j + 7) // 8
    out = out5.transpose(2, 4, 1, 3, 0).reshape(n_i, n_jt * 8, d_cols)
    return out[:, :n_j, :]


# speedup vs baseline: 85.0517x; 1.9029x over previous
import functools

import jax
import jax.numpy as jnp
from jax import lax
from jax.experimental import pallas as pl
from jax.experimental.pallas import tpu as pltpu
from jax.experimental.pallas import tpu_sc as plsc

_L = 16


def _build_sc_kernel(n_i, n_j, v_rows, d_cols):
    n_it = n_i // 128
    n_jt = (n_j + 7) // 8
    n_workers = 32
    it_per_w = n_it // n_workers
    grp = 4
    mesh = plsc.VectorSubcoreMesh(core_axis_name="c", subcore_axis_name="s")

    @functools.partial(
        pl.kernel,
        mesh=mesh,
        compiler_params=pltpu.CompilerParams(use_tc_tiling_on_sc=False,
                                             needs_layout_passes=False),
        out_type=jax.ShapeDtypeStruct((d_cols, n_jt, n_it, 8, 128),
                                      jnp.float32),
        scratch_types=[
            pltpu.VMEM((v_rows, _L), jnp.float32),
            pltpu.VMEM((v_rows, _L), jnp.float32),
            pltpu.VMEM((v_rows * _L,), jnp.float32),
            pltpu.VMEM((128 * n_j,), jnp.int32),
            pltpu.VMEM((d_cols, grp, 1, 8, 128), jnp.float32),
            pltpu.VMEM((d_cols, grp, 1, 8, 128), jnp.float32),
            pltpu.SemaphoreType.DMA,
            pltpu.SemaphoreType.DMA,
        ],
    )
    def k(idx_hbm, t1_hbm, t2_hbm, out_hbm, t1_v, t2_v, comb_v, idx_v,
          buf_a, buf_b, sem_a, sem_b):
        cid = lax.axis_index("c")
        sid = lax.axis_index("s")
        wid = sid * 2 + cid

        pltpu.sync_copy(t1_hbm, t1_v)
        pltpu.sync_copy(t2_hbm, t2_v)
        for r in range(v_rows):
            comb_v[pl.ds(r * _L, _L)] = t1_v[r, :] + t2_v[r, :]

        lanes100 = lax.iota(jnp.int32, _L) * n_j

        def do_group(it, jt0, njt, nj, buf):
            def col_chunk(jtl, jin, jglob, r):
                av = lanes100 + (r * _L * n_j + jglob)
                xi16 = plsc.load_gather(idx_v, [av]) * _L
                vals = [plsc.load_gather(comb_v, [xi16 + kk])
                        for kk in range(d_cols)]
                for kk in range(d_cols):
                    buf[kk, jtl, 0, jin, pl.ds(r * _L, _L)] = vals[kk]

            if njt > 1:
                for jtl in range(njt):
                    def jt_body(jin, carry, jtl=jtl):
                        jglob = (jt0 + jtl) * 8 + jin
                        for r in range(8):
                            col_chunk(jtl, jin, jglob, r)
                        return carry
                    lax.fori_loop(0, 8, jt_body, 0, unroll=False)
            else:
                def j_body(jl, carry):
                    for r in range(8):
                        col_chunk(0, jl, jt0 * 8 + jl, r)
                    return carry
                lax.fori_loop(0, nj, j_body, 0, unroll=False)
            descs = []
            for kk in range(d_cols):
                src = buf.at[kk, pl.ds(0, njt)]
                dst = out_hbm.at[kk, pl.ds(jt0, njt), pl.ds(it, 1)]
                descs.append(pltpu.async_copy(
                    src, dst, sem_a if buf is buf_a else sem_b))
            return descs

        def tile_body(t, carry):
            it = wid * it_per_w + t
            pltpu.sync_copy(idx_hbm.at[pl.ds(it * 128 * n_j, 128 * n_j)],
                            idx_v)
            d0 = do_group(it, 0, grp, 8 * grp, buf_a)
            d1 = do_group(it, grp, grp, 8 * grp, buf_b)
            for d in d0:
                d.wait()
            d2 = do_group(it, 2 * grp, grp, 8 * grp, buf_a)
            for d in d1:
                d.wait()
            d3 = do_group(it, 3 * grp, 1, n_j - 24 * grp, buf_b)
            for d in d2 + d3:
                d.wait()
            return carry

        lax.fori_loop(0, it_per_w, tile_body, 0, unroll=False)

    return k


@jax.jit
def kernel(x, emb1, emb2):
    n_i, n_j = x.shape
    v_rows, d_cols = emb1.shape
    xf = x.reshape(n_i * n_j).astype(jnp.int32)
    e1p = jnp.pad(emb1, ((0, 0), (0, _L - d_cols)))
    e2p = jnp.pad(emb2, ((0, 0), (0, _L - d_cols)))
    k = _build_sc_kernel(n_i, n_j, v_rows, d_cols)
    out5 = k(xf, e1p, e2p)
    n_jt = (n_j + 7) // 8
    out = out5.transpose(2, 4, 1, 3, 0).reshape(n_i, n_jt * 8, d_cols)
    return out[:, :n_j, :]


# speedup vs baseline: 85.3808x; 1.0039x over previous
import functools

import jax
import jax.numpy as jnp
from jax import lax
from jax.experimental import pallas as pl
from jax.experimental.pallas import tpu as pltpu
from jax.experimental.pallas import tpu_sc as plsc

_L = 16


def _build_sc_kernel(n_i, n_j, v_rows, d_cols):
    n_it = n_i // 128
    n_jt = (n_j + 7) // 8
    n_workers = 32
    it_per_w = n_it // n_workers
    grp = 4
    mesh = plsc.VectorSubcoreMesh(core_axis_name="c", subcore_axis_name="s")

    @functools.partial(
        pl.kernel,
        mesh=mesh,
        compiler_params=pltpu.CompilerParams(use_tc_tiling_on_sc=False,
                                             needs_layout_passes=False),
        out_type=jax.ShapeDtypeStruct((d_cols, n_jt, n_it, 8, 128),
                                      jnp.float32),
        scratch_types=[
            pltpu.VMEM((v_rows, _L), jnp.float32),
            pltpu.VMEM((v_rows, _L), jnp.float32),
            pltpu.VMEM((v_rows * _L,), jnp.float32),
            pltpu.VMEM((128 * n_j,), jnp.int32),
            pltpu.VMEM((d_cols, grp, 1, 8, 128), jnp.float32),
            pltpu.VMEM((d_cols, grp, 1, 8, 128), jnp.float32),
            pltpu.SemaphoreType.DMA,
            pltpu.SemaphoreType.DMA,
        ],
    )
    def k(idx_hbm, t1_hbm, t2_hbm, out_hbm, t1_v, t2_v, comb_v, idx_v,
          buf_a, buf_b, sem_a, sem_b):
        cid = lax.axis_index("c")
        sid = lax.axis_index("s")
        wid = sid * 2 + cid

        pltpu.sync_copy(t1_hbm, t1_v)
        pltpu.sync_copy(t2_hbm, t2_v)
        for r in range(v_rows):
            comb_v[pl.ds(r * _L, _L)] = t1_v[r, :] + t2_v[r, :]

        lanes100 = lax.iota(jnp.int32, _L) * n_j

        def do_group(it, jt0, njt, nj, buf):
            def col_chunk(jtl, jin, jglob, r):
                av = lanes100 + (r * _L * n_j + jglob)
                xi16 = plsc.load_gather(idx_v, [av]) * _L
                vals = [plsc.load_gather(comb_v, [xi16 + kk])
                        for kk in range(d_cols)]
                for kk in range(d_cols):
                    buf[kk, jtl, 0, jin, pl.ds(r * _L, _L)] = vals[kk]

            if njt > 1:
                for jtl in range(njt):
                    def jt_body(jin, carry, jtl=jtl):
                        jglob = (jt0 + jtl) * 8 + jin
                        for r in range(8):
                            col_chunk(jtl, jin, jglob, r)
                        return carry
                    lax.fori_loop(0, 8, jt_body, 0, unroll=False)
            else:
                def j_body(jl, carry):
                    for r in range(8):
                        col_chunk(0, jl, jt0 * 8 + jl, r)
                    return carry
                lax.fori_loop(0, nj, j_body, 0, unroll=False)
            src = buf.at[:, pl.ds(0, njt)]
            dst = out_hbm.at[:, pl.ds(jt0, njt), pl.ds(it, 1)]
            return [pltpu.async_copy(
                src, dst, sem_a if buf is buf_a else sem_b)]

        def tile_body(t, carry):
            it = wid * it_per_w + t
            pltpu.sync_copy(idx_hbm.at[pl.ds(it * 128 * n_j, 128 * n_j)],
                            idx_v)
            d0 = do_group(it, 0, grp, 8 * grp, buf_a)
            d1 = do_group(it, grp, grp, 8 * grp, buf_b)
            for d in d0:
                d.wait()
            d2 = do_group(it, 2 * grp, grp, 8 * grp, buf_a)
            for d in d1:
                d.wait()
            d3 = do_group(it, 3 * grp, 1, n_j - 24 * grp, buf_b)
            for d in d2 + d3:
                d.wait()
            return carry

        lax.fori_loop(0, it_per_w, tile_body, 0, unroll=False)

    return k


@jax.jit
def kernel(x, emb1, emb2):
    n_i, n_j = x.shape
    v_rows, d_cols = emb1.shape
    xf = x.reshape(n_i * n_j).astype(jnp.int32)
    e1p = jnp.pad(emb1, ((0, 0), (0, _L - d_cols)))
    e2p = jnp.pad(emb2, ((0, 0), (0, _L - d_cols)))
    k = _build_sc_kernel(n_i, n_j, v_rows, d_cols)
    out5 = k(xf, e1p, e2p)
    n_jt = (n_j + 7) // 8
    out = out5.transpose(2, 4, 1, 3, 0).reshape(n_i, n_jt * 8, d_cols)
    return out[:, :n_j, :]


# speedup vs baseline: 127.1797x; 1.4896x over previous
import functools

import jax
import jax.numpy as jnp
from jax import lax
from jax.experimental import pallas as pl
from jax.experimental.pallas import tpu as pltpu
from jax.experimental.pallas import tpu_sc as plsc

_L = 16


def _build_sc_kernel(n_i, n_j, v_rows, d_cols):
    n_it = n_i // 128
    n_jt = (n_j + 7) // 8
    n_workers = 32
    it_per_w = n_it // n_workers
    grp = 4
    rep = v_rows * _L
    mesh = plsc.VectorSubcoreMesh(core_axis_name="c", subcore_axis_name="s")

    @functools.partial(
        pl.kernel,
        mesh=mesh,
        compiler_params=pltpu.CompilerParams(use_tc_tiling_on_sc=False,
                                             needs_layout_passes=False),
        out_type=jax.ShapeDtypeStruct((d_cols, n_jt, n_it, 8, 128),
                                      jnp.float32),
        scratch_types=[
            pltpu.VMEM((v_rows, _L), jnp.float32),
            pltpu.VMEM((v_rows, _L), jnp.float32),
            pltpu.VMEM((v_rows * _L,), jnp.float32),
            pltpu.VMEM((d_cols * rep,), jnp.float32),
            pltpu.VMEM((128 * n_j,), jnp.int32),
            pltpu.VMEM((d_cols, grp, 1, 8, 128), jnp.float32),
            pltpu.VMEM((d_cols, grp, 1, 8, 128), jnp.float32),
            pltpu.SemaphoreType.DMA,
            pltpu.SemaphoreType.DMA,
        ],
    )
    def k(idx_hbm, t1_hbm, t2_hbm, out_hbm, t1_v, t2_v, comb_v, rep_v,
          idx_v, buf_a, buf_b, sem_a, sem_b):
        cid = lax.axis_index("c")
        sid = lax.axis_index("s")
        wid = sid * 2 + cid

        lanes = lax.iota(jnp.int32, _L)
        lanes0 = lanes * 0
        lanes100 = lanes * n_j

        pltpu.sync_copy(t1_hbm, t1_v)
        pltpu.sync_copy(t2_hbm, t2_v)
        for r in range(v_rows):
            comb_v[pl.ds(r * _L, _L)] = t1_v[r, :] + t2_v[r, :]
        for kk in range(d_cols):
            for v in range(v_rows):
                val = plsc.load_gather(comb_v, [lanes0 + (v * _L + kk)])
                rep_v[pl.ds(kk * rep + v * _L, _L)] = val

        def do_group(it, jt0, njt, nj, buf):
            def col_chunk(jtl, jin, jglob, r):
                av = lanes100 + (r * _L * n_j + jglob)
                xi = plsc.load_gather(idx_v, [av])
                xi16 = xi * _L + lanes
                vals = [plsc.load_gather(rep_v, [xi16 + kk * rep])
                        for kk in range(d_cols)]
                for kk in range(d_cols):
                    buf[kk, jtl, 0, jin, pl.ds(r * _L, _L)] = vals[kk]

            if njt > 1:
                for jtl in range(njt):
                    def jt_body(jin, carry, jtl=jtl):
                        jglob = (jt0 + jtl) * 8 + jin
                        for r in range(8):
                            col_chunk(jtl, jin, jglob, r)
                        return carry
                    lax.fori_loop(0, 8, jt_body, 0, unroll=False)
            else:
                def j_body(jl, carry):
                    for r in range(8):
                        col_chunk(0, jl, jt0 * 8 + jl, r)
                    return carry
                lax.fori_loop(0, nj, j_body, 0, unroll=False)
            src = buf.at[:, pl.ds(0, njt)]
            dst = out_hbm.at[:, pl.ds(jt0, njt), pl.ds(it, 1)]
            return [pltpu.async_copy(
                src, dst, sem_a if buf is buf_a else sem_b)]

        def tile_body(t, carry):
            it = wid * it_per_w + t
            pltpu.sync_copy(idx_hbm.at[pl.ds(it * 128 * n_j, 128 * n_j)],
                            idx_v)
            d0 = do_group(it, 0, grp, 8 * grp, buf_a)
            d1 = do_group(it, grp, grp, 8 * grp, buf_b)
            for d in d0:
                d.wait()
            d2 = do_group(it, 2 * grp, grp, 8 * grp, buf_a)
            for d in d1:
                d.wait()
            d3 = do_group(it, 3 * grp, 1, n_j - 24 * grp, buf_b)
            for d in d2 + d3:
                d.wait()
            return carry

        lax.fori_loop(0, it_per_w, tile_body, 0, unroll=False)

    return k


@jax.jit
def kernel(x, emb1, emb2):
    n_i, n_j = x.shape
    v_rows, d_cols = emb1.shape
    xf = x.reshape(n_i * n_j).astype(jnp.int32)
    e1p = jnp.pad(emb1, ((0, 0), (0, _L - d_cols)))
    e2p = jnp.pad(emb2, ((0, 0), (0, _L - d_cols)))
    k = _build_sc_kernel(n_i, n_j, v_rows, d_cols)
    out5 = k(xf, e1p, e2p)
    n_jt = (n_j + 7) // 8
    out = out5.transpose(2, 4, 1, 3, 0).reshape(n_i, n_jt * 8, d_cols)
    return out[:, :n_j, :]


# speedup vs baseline: 128.6777x; 1.0118x over previous
import functools

import jax
import jax.numpy as jnp
from jax import lax
from jax.experimental import pallas as pl
from jax.experimental.pallas import tpu as pltpu
from jax.experimental.pallas import tpu_sc as plsc

_L = 16


def _build_sc_kernel(n_i, n_j, v_rows, d_cols):
    n_it = n_i // 128
    n_jt = (n_j + 7) // 8
    n_workers = 32
    it_per_w = n_it // n_workers
    grp = 4
    stride = _L + 1
    mesh = plsc.VectorSubcoreMesh(core_axis_name="c", subcore_axis_name="s")

    @functools.partial(
        pl.kernel,
        mesh=mesh,
        compiler_params=pltpu.CompilerParams(use_tc_tiling_on_sc=False,
                                             needs_layout_passes=False),
        out_type=jax.ShapeDtypeStruct((d_cols, n_jt, n_it, 8, 128),
                                      jnp.float32),
        scratch_types=[
            pltpu.VMEM((v_rows, _L), jnp.float32),
            pltpu.VMEM((v_rows, _L), jnp.float32),
            pltpu.VMEM((v_rows * stride,), jnp.float32),
            pltpu.VMEM((128 * n_j,), jnp.int32),
            pltpu.VMEM((d_cols, grp, 1, 8, 128), jnp.float32),
            pltpu.VMEM((d_cols, grp, 1, 8, 128), jnp.float32),
            pltpu.SemaphoreType.DMA,
            pltpu.SemaphoreType.DMA,
        ],
    )
    def k(idx_hbm, t1_hbm, t2_hbm, out_hbm, t1_v, t2_v, rep_v,
          idx_v, buf_a, buf_b, sem_a, sem_b):
        cid = lax.axis_index("c")
        sid = lax.axis_index("s")
        wid = sid * 2 + cid

        lanes = lax.iota(jnp.int32, _L)
        lanes0 = lanes * 0
        lanes100 = lanes * n_j

        pltpu.sync_copy(t1_hbm, t1_v)
        pltpu.sync_copy(t2_hbm, t2_v)
        for v in range(v_rows):
            rep_v[pl.ds(v * stride, _L)] = t1_v[v, :] + t2_v[v, :]

        def do_group(it, jt0, njt, nj, buf):
            def col_chunk(jtl, jin, jglob, r):
                av = lanes100 + (r * _L * n_j + jglob)
                xi = plsc.load_gather(idx_v, [av])
                xi17 = xi * stride
                vals = [plsc.load_gather(rep_v, [xi17 + kk])
                        for kk in range(d_cols)]
                for kk in range(d_cols):
                    buf[kk, jtl, 0, jin, pl.ds(r * _L, _L)] = vals[kk]

            if njt > 1:
                for jtl in range(njt):
                    def jt_body(jin, carry, jtl=jtl):
                        jglob = (jt0 + jtl) * 8 + jin
                        for r in range(8):
                            col_chunk(jtl, jin, jglob, r)
                        return carry
                    lax.fori_loop(0, 8, jt_body, 0, unroll=False)
            else:
                def j_body(jl, carry):
                    for r in range(8):
                        col_chunk(0, jl, jt0 * 8 + jl, r)
                    return carry
                lax.fori_loop(0, nj, j_body, 0, unroll=False)
            src = buf.at[:, pl.ds(0, njt)]
            dst = out_hbm.at[:, pl.ds(jt0, njt), pl.ds(it, 1)]
            return [pltpu.async_copy(
                src, dst, sem_a if buf is buf_a else sem_b)]

        def tile_body(t, carry):
            it = wid * it_per_w + t
            pltpu.sync_copy(idx_hbm.at[pl.ds(it * 128 * n_j, 128 * n_j)],
                            idx_v)
            d0 = do_group(it, 0, grp, 8 * grp, buf_a)
            d1 = do_group(it, grp, grp, 8 * grp, buf_b)
            for d in d0:
                d.wait()
            d2 = do_group(it, 2 * grp, grp, 8 * grp, buf_a)
            for d in d1:
                d.wait()
            d3 = do_group(it, 3 * grp, 1, n_j - 24 * grp, buf_b)
            for d in d2 + d3:
                d.wait()
            return carry

        lax.fori_loop(0, it_per_w, tile_body, 0, unroll=False)

    return k


@jax.jit
def kernel(x, emb1, emb2):
    n_i, n_j = x.shape
    v_rows, d_cols = emb1.shape
    xf = x.reshape(n_i * n_j).astype(jnp.int32)
    e1p = jnp.pad(emb1, ((0, 0), (0, _L - d_cols)))
    e2p = jnp.pad(emb2, ((0, 0), (0, _L - d_cols)))
    k = _build_sc_kernel(n_i, n_j, v_rows, d_cols)
    out5 = k(xf, e1p, e2p)
    n_jt = (n_j + 7) // 8
    out = out5.transpose(2, 4, 1, 3, 0).reshape(n_i, n_jt * 8, d_cols)
    return out[:, :n_j, :]


# speedup vs baseline: 131.7749x; 1.0241x over previous
import functools

import jax
import jax.numpy as jnp
from jax import lax
from jax.experimental import pallas as pl
from jax.experimental.pallas import tpu as pltpu
from jax.experimental.pallas import tpu_sc as plsc

_L = 16


def _build_sc_kernel(n_i, n_j, v_rows, d_cols):
    n_it = n_i // 128
    n_jt = (n_j + 7) // 8
    n_workers = 32
    it_per_w = n_it // n_workers
    grp = 2
    stride = _L + 1
    mesh = plsc.VectorSubcoreMesh(core_axis_name="c", subcore_axis_name="s")

    @functools.partial(
        pl.kernel,
        mesh=mesh,
        compiler_params=pltpu.CompilerParams(use_tc_tiling_on_sc=False,
                                             needs_layout_passes=False),
        out_type=jax.ShapeDtypeStruct((d_cols, n_jt, n_it, 8, 128),
                                      jnp.float32),
        scratch_types=[
            pltpu.VMEM((v_rows, _L), jnp.float32),
            pltpu.VMEM((v_rows, _L), jnp.float32),
            pltpu.VMEM((v_rows * stride,), jnp.float32),
            pltpu.VMEM((128 * n_j,), jnp.int32),
            pltpu.VMEM((d_cols, grp, 1, 8, 128), jnp.float32),
            pltpu.VMEM((d_cols, grp, 1, 8, 128), jnp.float32),
            pltpu.SemaphoreType.DMA,
            pltpu.SemaphoreType.DMA,
        ],
    )
    def k(idx_hbm, t1_hbm, t2_hbm, out_hbm, t1_v, t2_v, rep_v,
          idx_v, buf_a, buf_b, sem_a, sem_b):
        cid = lax.axis_index("c")
        sid = lax.axis_index("s")
        wid = sid * 2 + cid

        lanes = lax.iota(jnp.int32, _L)
        lanes0 = lanes * 0
        lanes100 = lanes * n_j

        pltpu.sync_copy(t1_hbm, t1_v)
        pltpu.sync_copy(t2_hbm, t2_v)
        for v in range(v_rows):
            rep_v[pl.ds(v * stride, _L)] = t1_v[v, :] + t2_v[v, :]

        def do_group(it, jt0, njt, nj, buf):
            def col_chunk(jtl, jin, jglob, r):
                av = lanes100 + (r * _L * n_j + jglob)
                xi = plsc.load_gather(idx_v, [av])
                xi17 = xi * stride
                vals = [plsc.load_gather(rep_v, [xi17 + kk])
                        for kk in range(d_cols)]
                for kk in range(d_cols):
                    buf[kk, jtl, 0, jin, pl.ds(r * _L, _L)] = vals[kk]

            if njt > 1:
                for jtl in range(njt):
                    def jt_body(jin, carry, jtl=jtl):
                        jglob = (jt0 + jtl) * 8 + jin
                        for r in range(8):
                            col_chunk(jtl, jin, jglob, r)
                        return carry
                    lax.fori_loop(0, 8, jt_body, 0, unroll=False)
            else:
                def j_body(jl, carry):
                    for r in range(8):
                        col_chunk(0, jl, jt0 * 8 + jl, r)
                    return carry
                lax.fori_loop(0, nj, j_body, 0, unroll=False)
            src = buf.at[:, pl.ds(0, njt)]
            dst = out_hbm.at[:, pl.ds(jt0, njt), pl.ds(it, 1)]
            return [pltpu.async_copy(
                src, dst, sem_a if buf is buf_a else sem_b)]

        groups = [(jt0, min(grp, n_jt - jt0))
                  for jt0 in range(0, n_jt, grp)]
        bufs = (buf_a, buf_b)

        def tile_body(t, carry):
            it = wid * it_per_w + t
            pltpu.sync_copy(idx_hbm.at[pl.ds(it * 128 * n_j, 128 * n_j)],
                            idx_v)
            prev = [None, None]
            for gi, (jt0, njt) in enumerate(groups):
                b = gi % 2
                if prev[b] is not None:
                    for d in prev[b]:
                        d.wait()
                nj = min(8 * njt, n_j - 8 * jt0)
                prev[b] = do_group(it, jt0, njt, nj, bufs[b])
            for pr in prev:
                if pr is not None:
                    for d in pr:
                        d.wait()
            return carry

        lax.fori_loop(0, it_per_w, tile_body, 0, unroll=False)

    return k


@jax.jit
def kernel(x, emb1, emb2):
    n_i, n_j = x.shape
    v_rows, d_cols = emb1.shape
    xf = x.reshape(n_i * n_j).astype(jnp.int32)
    e1p = jnp.pad(emb1, ((0, 0), (0, _L - d_cols)))
    e2p = jnp.pad(emb2, ((0, 0), (0, _L - d_cols)))
    k = _build_sc_kernel(n_i, n_j, v_rows, d_cols)
    out5 = k(xf, e1p, e2p)
    n_jt = (n_j + 7) // 8
    out = out5.transpose(2, 4, 1, 3, 0).reshape(n_i, n_jt * 8, d_cols)
    return out[:, :n_j, :]


# speedup vs baseline: 133.1222x; 1.0102x over previous
import functools

import jax
import jax.numpy as jnp
from jax import lax
from jax.experimental import pallas as pl
from jax.experimental.pallas import tpu as pltpu
from jax.experimental.pallas import tpu_sc as plsc

_L = 16


def _build_sc_kernel(n_i, n_j, v_rows, d_cols):
    n_it = n_i // 128
    n_jt = (n_j + 7) // 8
    n_workers = 32
    it_per_w = n_it // n_workers
    grp = 1
    stride = _L + 1
    mesh = plsc.VectorSubcoreMesh(core_axis_name="c", subcore_axis_name="s")

    @functools.partial(
        pl.kernel,
        mesh=mesh,
        compiler_params=pltpu.CompilerParams(use_tc_tiling_on_sc=False,
                                             needs_layout_passes=False),
        out_type=jax.ShapeDtypeStruct((d_cols, n_jt, n_it, 8, 128),
                                      jnp.float32),
        scratch_types=[
            pltpu.VMEM((v_rows, _L), jnp.float32),
            pltpu.VMEM((v_rows, _L), jnp.float32),
            pltpu.VMEM((v_rows * stride,), jnp.float32),
            pltpu.VMEM((128 * n_j,), jnp.int32),
            pltpu.VMEM((d_cols, grp, 1, 8, 128), jnp.float32),
            pltpu.VMEM((d_cols, grp, 1, 8, 128), jnp.float32),
            pltpu.SemaphoreType.DMA,
            pltpu.SemaphoreType.DMA,
        ],
    )
    def k(idx_hbm, t1_hbm, t2_hbm, out_hbm, t1_v, t2_v, rep_v,
          idx_v, buf_a, buf_b, sem_a, sem_b):
        cid = lax.axis_index("c")
        sid = lax.axis_index("s")
        wid = sid * 2 + cid

        lanes = lax.iota(jnp.int32, _L)
        lanes0 = lanes * 0
        lanes100 = lanes * n_j

        pltpu.sync_copy(t1_hbm, t1_v)
        pltpu.sync_copy(t2_hbm, t2_v)
        for v in range(v_rows):
            rep_v[pl.ds(v * stride, _L)] = t1_v[v, :] + t2_v[v, :]

        def do_group(it, jt0, njt, nj, buf):
            def col_chunk(jtl, jin, jglob, r):
                av = lanes100 + (r * _L * n_j + jglob)
                xi = plsc.load_gather(idx_v, [av])
                xi17 = xi * stride
                vals = [plsc.load_gather(rep_v, [xi17 + kk])
                        for kk in range(d_cols)]
                for kk in range(d_cols):
                    buf[kk, jtl, 0, jin, pl.ds(r * _L, _L)] = vals[kk]

            if njt > 1:
                for jtl in range(njt):
                    def jt_body(jin, carry, jtl=jtl):
                        jglob = (jt0 + jtl) * 8 + jin
                        for r in range(8):
                            col_chunk(jtl, jin, jglob, r)
                        return carry
                    lax.fori_loop(0, 8, jt_body, 0, unroll=False)
            else:
                def j_body(jl, carry):
                    for r in range(8):
                        col_chunk(0, jl, jt0 * 8 + jl, r)
                    return carry
                lax.fori_loop(0, nj, j_body, 0, unroll=False)
            src = buf.at[:, pl.ds(0, njt)]
            dst = out_hbm.at[:, pl.ds(jt0, njt), pl.ds(it, 1)]
            return [pltpu.async_copy(
                src, dst, sem_a if buf is buf_a else sem_b)]

        groups = [(jt0, min(grp, n_jt - jt0))
                  for jt0 in range(0, n_jt, grp)]
        bufs = (buf_a, buf_b)

        def tile_body(t, carry):
            it = wid * it_per_w + t
            pltpu.sync_copy(idx_hbm.at[pl.ds(it * 128 * n_j, 128 * n_j)],
                            idx_v)
            prev = [None, None]
            for gi, (jt0, njt) in enumerate(groups):
                b = gi % 2
                if prev[b] is not None:
                    for d in prev[b]:
                        d.wait()
                nj = min(8 * njt, n_j - 8 * jt0)
                prev[b] = do_group(it, jt0, njt, nj, bufs[b])
            for pr in prev:
                if pr is not None:
                    for d in pr:
                        d.wait()
            return carry

        lax.fori_loop(0, it_per_w, tile_body, 0, unroll=False)

    return k


@jax.jit
def kernel(x, emb1, emb2):
    n_i, n_j = x.shape
    v_rows, d_cols = emb1.shape
    xf = x.reshape(n_i * n_j).astype(jnp.int32)
    e1p = jnp.pad(emb1, ((0, 0), (0, _L - d_cols)))
    e2p = jnp.pad(emb2, ((0, 0), (0, _L - d_cols)))
    k = _build_sc_kernel(n_i, n_j, v_rows, d_cols)
    out5 = k(xf, e1p, e2p)
    n_jt = (n_j + 7) // 8
    out = out5.transpose(2, 4, 1, 3, 0).reshape(n_i, n_jt * 8, d_cols)
    return out[:, :n_j, :]
